# speedup vs baseline: 4.8650x; 4.8650x over previous
import functools

import jax
import jax.numpy as jnp
from jax.experimental import pallas as pl
from jax.experimental.pallas import tpu as pltpu


def _rnn_step(e_ref, wih_ref, whh_ref, bh_ref, wun_ref, bun_ref,
              y_ref, s_ref):
    t = pl.program_id(0)

    @pl.when(t == 0)
    def _():
        s_ref[...] = jnp.zeros_like(s_ref)

    e_t = e_ref[0]
    s = s_ref[...]
    h = (jnp.dot(e_t, wih_ref[...], preferred_element_type=jnp.float32)
         + jnp.dot(s, whh_ref[...], preferred_element_type=jnp.float32)
         + bh_ref[...])
    s = jnp.tanh(h)
    s_ref[...] = s
    y_ref[0] = (jnp.dot(s, wun_ref[...], preferred_element_type=jnp.float32)
                + bun_ref[...])


def _rnn_unembed(e, wih_t, whh_t, bh, wun_t, bun, *, interpret=False):
    T, B, E = e.shape
    A = wun_t.shape[1]
    return pl.pallas_call(
        _rnn_step,
        grid=(T,),
        in_specs=[
            pl.BlockSpec((1, B, E), lambda t: (t, 0, 0)),
            pl.BlockSpec((E, E), lambda t: (0, 0)),
            pl.BlockSpec((E, E), lambda t: (0, 0)),
            pl.BlockSpec((1, E), lambda t: (0, 0)),
            pl.BlockSpec((E, A), lambda t: (0, 0)),
            pl.BlockSpec((1, A), lambda t: (0, 0)),
        ],
        out_specs=pl.BlockSpec((1, B, A), lambda t: (t, 0, 0)),
        out_shape=jax.ShapeDtypeStruct((T, B, A), jnp.float32),
        scratch_shapes=[pltpu.VMEM((B, E), jnp.float32)],
        compiler_params=pltpu.CompilerParams(
            dimension_semantics=("arbitrary",),
        ),
        interpret=interpret,
    )(e, wih_t, whh_t, bh, wun_t, bun)


def kernel(x, trainable, embed_table, W_ih, W_hh, b_h, W_un, b_un):
    B, T = x.shape
    E = embed_table.shape[1]
    A = W_un.shape[0]
    idx = x.T.reshape(-1)
    e = jnp.take(embed_table, idx, axis=0).reshape(T, B, E)
    y = _rnn_unembed(e, W_ih.T, W_hh.T, b_h.reshape(1, E),
                     W_un.T, b_un.reshape(1, A))
    return y.reshape(B, T, A)


# speedup vs baseline: 10.6324x; 2.1855x over previous
import functools

import jax
import jax.numpy as jnp
from jax import lax
from jax.experimental import pallas as pl
from jax.experimental.pallas import tpu as pltpu
from jax.experimental.pallas import tpu_sc as plsc

_CHUNK = 128


def _gather_body(n_chunks, table_hbm, idx_hbm, out_hbm,
                 idx_v, rows_v, sem0, sem1):
    nc = plsc.get_sparse_core_info().num_cores
    wid = lax.axis_index("s") * nc + lax.axis_index("c")
    rows_per_w = n_chunks * _CHUNK
    base = wid * rows_per_w
    pltpu.sync_copy(idx_hbm.at[wid], idx_v)

    def gather(j, buf, sem):
        pltpu.make_async_copy(
            table_hbm.at[idx_v.at[j]], rows_v.at[buf], sem).start()

    def wait(j, buf, sem):
        pltpu.make_async_copy(
            table_hbm.at[idx_v.at[j]], rows_v.at[buf], sem).wait()

    gather(0, 0, sem0)
    gather(1, 1, sem1)

    def body(g, carry):
        c0 = 2 * g
        wait(c0, 0, sem0)
        pltpu.sync_copy(rows_v.at[0], out_hbm.at[pl.ds(base + c0 * _CHUNK, _CHUNK)])

        @pl.when(c0 + 2 < n_chunks)
        def _():
            gather(c0 + 2, 0, sem0)

        wait(c0 + 1, 1, sem1)
        pltpu.sync_copy(rows_v.at[1],
                        out_hbm.at[pl.ds(base + (c0 + 1) * _CHUNK, _CHUNK)])

        @pl.when(c0 + 3 < n_chunks)
        def _():
            gather(c0 + 3, 1, sem1)

        return carry

    lax.fori_loop(0, n_chunks // 2, body, 0)


def _sc_gather(table, idx, n_rows, E):
    nw, n_chunks, _ = idx.shape
    mesh = plsc.VectorSubcoreMesh(core_axis_name="c", subcore_axis_name="s")
    return pl.kernel(
        functools.partial(_gather_body, n_chunks),
        out_type=jax.ShapeDtypeStruct((n_rows, E), jnp.float32),
        mesh=mesh,
        scratch_types=[
            pltpu.VMEM((n_chunks, _CHUNK), jnp.int32),
            pltpu.VMEM((2, _CHUNK, E), jnp.float32),
            pltpu.SemaphoreType.DMA,
            pltpu.SemaphoreType.DMA,
        ],
        compiler_params=pltpu.CompilerParams(use_tc_tiling_on_sc=False),
    )(table, idx)


def _rnn_step(e_ref, wih_ref, whh_ref, bh_ref, wun_ref, bun_ref,
              y_ref, s_ref):
    t = pl.program_id(0)

    @pl.when(t == 0)
    def _():
        s_ref[...] = jnp.zeros_like(s_ref)

    e_t = e_ref[0]
    s = s_ref[...]
    h = (jnp.dot(e_t, wih_ref[...], preferred_element_type=jnp.float32)
         + jnp.dot(s, whh_ref[...], preferred_element_type=jnp.float32)
         + bh_ref[...])
    s = jnp.tanh(h)
    s_ref[...] = s
    y_ref[0] = (jnp.dot(s, wun_ref[...], preferred_element_type=jnp.float32)
                + bun_ref[...])


def _rnn_unembed(e, wih_t, whh_t, bh, wun_t, bun, *, interpret=False):
    T, B, E = e.shape
    A = wun_t.shape[1]
    return pl.pallas_call(
        _rnn_step,
        grid=(T,),
        in_specs=[
            pl.BlockSpec((1, B, E), lambda t: (t, 0, 0)),
            pl.BlockSpec((E, E), lambda t: (0, 0)),
            pl.BlockSpec((E, E), lambda t: (0, 0)),
            pl.BlockSpec((1, E), lambda t: (0, 0)),
            pl.BlockSpec((E, A), lambda t: (0, 0)),
            pl.BlockSpec((1, A), lambda t: (0, 0)),
        ],
        out_specs=pl.BlockSpec((1, B, A), lambda t: (t, 0, 0)),
        out_shape=jax.ShapeDtypeStruct((T, B, A), jnp.float32),
        scratch_shapes=[pltpu.VMEM((B, E), jnp.float32)],
        compiler_params=pltpu.CompilerParams(
            dimension_semantics=("arbitrary",),
        ),
        interpret=interpret,
    )(e, wih_t, whh_t, bh, wun_t, bun)


def kernel(x, trainable, embed_table, W_ih, W_hh, b_h, W_un, b_un):
    B, T = x.shape
    E = embed_table.shape[1]
    A = W_un.shape[0]
    n_rows = B * T
    nw = 32
    idx = x.T.reshape(nw, n_rows // (nw * _CHUNK), _CHUNK)
    e = _sc_gather(embed_table, idx, n_rows, E).reshape(T, B, E)
    y = _rnn_unembed(e, W_ih.T, W_hh.T, b_h.reshape(1, E),
                     W_un.T, b_un.reshape(1, A))
    return y.reshape(B, T, A)
